# baseline (speedup 1.0000x reference)
import functools

import jax
import jax.numpy as jnp
from jax import lax
from jax.experimental import pallas as pl
from jax.experimental.pallas import tpu as pltpu
from jax.experimental.pallas import tpu_sc as plsc

NC = 2
NS = 16
NW = NC * NS
CHUNK = 128


def _make_gather(B, V, D):
  b_per_w = B // NW
  n_chunk = b_per_w // CHUNK
  mesh = plsc.VectorSubcoreMesh(core_axis_name="c", subcore_axis_name="s")

  @functools.partial(
      pl.kernel,
      mesh=mesh,
      out_type=jax.ShapeDtypeStruct((B, D), jnp.float32),
      scratch_types=[
          pltpu.VMEM((n_chunk, CHUNK), jnp.int32),
          pltpu.VMEM((b_per_w, D), jnp.float32),
          pltpu.SemaphoreType.DMA,
      ],
      compiler_params=pltpu.CompilerParams(use_tc_tiling_on_sc=False),
  )
  def k(table_hbm, idx_hbm, out_hbm, idx_v, rows_v, sem):
    wid = lax.axis_index("s") * NC + lax.axis_index("c")
    pltpu.sync_copy(idx_hbm.at[pl.ds(wid * n_chunk, n_chunk)], idx_v)
    copies = []
    for j in range(n_chunk):
      copies.append(
          pltpu.async_copy(
              table_hbm.at[idx_v.at[j]],
              rows_v.at[pl.ds(j * CHUNK, CHUNK)],
              sem,
          ))
    for c in copies:
      c.wait()
    pltpu.sync_copy(rows_v, out_hbm.at[pl.ds(wid * b_per_w, b_per_w)])

  return k


def kernel(index, head_e):
  B = index.shape[0]
  V, D = head_e.shape
  idx2d = index.astype(jnp.int32).reshape(B // CHUNK, CHUNK)
  return _make_gather(B, V, D)(head_e, idx2d)


# speedup vs baseline: 1.8592x; 1.8592x over previous
import functools

import jax
import jax.numpy as jnp
from jax import lax
from jax.experimental import pallas as pl
from jax.experimental.pallas import tpu as pltpu
from jax.experimental.pallas import tpu_sc as plsc

NC = 2
NS = 16
NW = NC * NS
L = 16

V = 1000000
D = 64
B = 16384
TILE_E = 128
NTJ = V // TILE_E
TAIL_BASE = NTJ * TILE_E
TAIL_N = V - TAIL_BASE
TPW = -(-NTJ // NW)
RING = 16


def _scalar(x):
  return jnp.max(x)


def _bcast_lane(vec, pos_v):
  dnums = lax.GatherDimensionNumbers(
      offset_dims=(), collapsed_slice_dims=(0,), start_index_map=(0,))
  return lax.gather(
      vec, pos_v.reshape(L, 1), dnums, (1,),
      mode=lax.GatherScatterMode.PROMISE_IN_BOUNDS)


def _emit_extract(tb, is_tail, tj_s, he, hb, idx_cap, stage, cnt_ref, out_hbm,
                  sem_o, dummy_v):
  tj_v = jnp.broadcast_to(tj_s, (L,))

  def group_body(g, _):
    ev = he[pl.ds(g * L, L)]
    m = (lax.shift_right_logical(ev, 7) == tj_v) & (ev >= 0)

    @pl.when(jnp.max(m.astype(jnp.int32)) > 0)
    def _():
      bv = hb[pl.ds(g * L, L)]

      def w_cond(carry):
        mm = carry
        return jnp.max(mm.astype(jnp.int32)) > 0

      def w_body(carry):
        mm = carry
        pos_s = _scalar(plsc.all_reduce_ffs(mm))
        pos_v = jnp.broadcast_to(pos_s, (L,))
        e_all = _bcast_lane(ev, pos_v)
        b_all = _bcast_lane(bv, pos_v)
        if is_tail:
          el = e_all - TAIL_BASE
        else:
          el = e_all & (TILE_E - 1)
        c_s = _scalar(cnt_ref[...])
        slot = lax.rem(c_s, RING)

        @pl.when(c_s >= RING)
        def _():
          pltpu.make_async_copy(
              out_hbm.at[pl.ds(0, D)], dummy_v, sem_o).wait()

        slot_v = jnp.broadcast_to(slot * D, (L,))
        for k in range(D // L):
          f_v = lax.iota(jnp.int32, L) + (k * L)
          if is_tail:
            vals = plsc.load_gather(tb, [f_v * TAIL_N + el])
          else:
            vals = plsc.load_gather(tb, [f_v, el])
          plsc.store_scatter(stage, [slot_v + f_v], vals)
        b_s = _scalar(b_all)
        pltpu.async_copy(
            stage.at[pl.ds(slot * D, D)],
            out_hbm.at[pl.ds(b_s * D, D)],
            sem_o,
        )
        cnt_ref[...] = jnp.broadcast_to(c_s + 1, (L,))
        lane = lax.iota(jnp.int32, L)
        return mm & (lane != jnp.broadcast_to(pos_s, (L,)))

      lax.while_loop(w_cond, w_body, m)

    return 0

  n_groups = lax.div(idx_cap + (L - 1), L)
  lax.fori_loop(0, n_groups, group_body, 0)


def _make_kernel():
  mesh = plsc.VectorSubcoreMesh(core_axis_name="c", subcore_axis_name="s")
  hit_cap = B + L

  @functools.partial(
      pl.kernel,
      mesh=mesh,
      out_type=jax.ShapeDtypeStruct((B * D,), jnp.float32),
      scratch_types=[
          pltpu.VMEM((B,), jnp.int32),
          pltpu.VMEM((hit_cap,), jnp.int32),
          pltpu.VMEM((hit_cap,), jnp.int32),
          pltpu.VMEM((D, TILE_E), jnp.float32),
          pltpu.VMEM((D, TILE_E), jnp.float32),
          pltpu.VMEM((D * TAIL_N,), jnp.float32),
          pltpu.VMEM((RING * D,), jnp.float32),
          pltpu.VMEM((D,), jnp.float32),
          pltpu.VMEM((L,), jnp.int32),
          pltpu.VMEM((L,), jnp.int32),
          pltpu.SemaphoreType.DMA,
          pltpu.SemaphoreType.DMA,
          pltpu.SemaphoreType.DMA,
      ],
      compiler_params=pltpu.CompilerParams(needs_layout_passes=False),
  )
  def k(tab_hbm, tail_hbm, idx_hbm, out_hbm, idx_v, he, hb, tb0, tb1, tailb,
        stage, dummy_v, pos_ref, cnt_ref, sem_i, sem_t, sem_o):
    wid = lax.axis_index("s") * NC + lax.axis_index("c")
    lo = wid * TPW
    nloc = jnp.minimum(TPW, NTJ - lo)
    is_last = wid == NW - 1
    hi = jnp.where(is_last, NTJ + 1, lo + TPW)
    lo_v = jnp.broadcast_to(lo, (L,))
    hi_v = jnp.broadcast_to(hi, (L,))

    pltpu.sync_copy(idx_hbm, idx_v)
    pos_ref[...] = jnp.zeros((L,), jnp.int32)
    cnt_ref[...] = jnp.zeros((L,), jnp.int32)

    def fill_body(i, _):
      he[pl.ds(i * L, L)] = jnp.broadcast_to(jnp.int32(-1), (L,))
      return 0

    lax.fori_loop(0, hit_cap // L, fill_body, 0)

    def scan_body(i, _):
      v = idx_v[pl.ds(i * L, L)]
      tj = lax.shift_right_logical(v, 7)
      m = (tj >= lo_v) & (tj < hi_v)
      pos_v = pos_ref[...]
      c = plsc.cumsum(m.astype(jnp.int32))
      target = pos_v + c - 1
      plsc.store_scatter(he, [target], v, mask=m)
      bvec = lax.iota(jnp.int32, L) + (i * L)
      plsc.store_scatter(hb, [target], bvec, mask=m)
      pos_ref[...] = pos_v + jnp.broadcast_to(_scalar(c), (L,))
      return 0

    lax.fori_loop(0, B // L, scan_body, 0)
    idx_cap = _scalar(pos_ref[...])

    @pl.when(nloc > 0)
    def _():
      pltpu.async_copy(
          tab_hbm.at[:, pl.ds(lo * TILE_E, TILE_E)], tb0, sem_t)

    def outer_body(o, _):
      for kk in range(2):
        i = o * 2 + kk
        tb_cur = tb0 if kk == 0 else tb1
        tb_nxt = tb1 if kk == 0 else tb0

        @pl.when(i < nloc)
        def _():
          @pl.when(i + 1 < nloc)
          def _():
            pltpu.async_copy(
                tab_hbm.at[:, pl.ds((lo + i + 1) * TILE_E, TILE_E)],
                tb_nxt, sem_t)

          pltpu.make_async_copy(
              tab_hbm.at[:, pl.ds(0, TILE_E)], tb_cur, sem_t).wait()
          _emit_extract(tb_cur, False, lo + i, he, hb, idx_cap, stage,
                        cnt_ref, out_hbm, sem_o, dummy_v)

      return 0

    lax.fori_loop(0, (TPW + 1) // 2, outer_body, 0)

    @pl.when(is_last)
    def _():
      pltpu.async_copy(tail_hbm, tailb, sem_t)
      pltpu.make_async_copy(tail_hbm, tailb, sem_t).wait()
      _emit_extract(tailb, True, jnp.int32(NTJ), he, hb, idx_cap, stage,
                    cnt_ref, out_hbm, sem_o, dummy_v)

    c_tot = _scalar(cnt_ref[...])

    def drain_body(i, _):
      pltpu.make_async_copy(out_hbm.at[pl.ds(0, D)], dummy_v, sem_o).wait()
      return 0

    lax.fori_loop(0, jnp.minimum(c_tot, RING), drain_body, 0)

  return k


def kernel(index, head_e):
  tab_t = head_e.T
  tail = head_e[TAIL_BASE:].T.reshape(D * TAIL_N)
  flat = _make_kernel()(tab_t, tail, index.astype(jnp.int32))
  return flat.reshape(B, D)


# speedup vs baseline: 3.4254x; 1.8424x over previous
import functools

import jax
import jax.numpy as jnp
from jax import lax
from jax.experimental import pallas as pl
from jax.experimental.pallas import tpu as pltpu
from jax.experimental.pallas import tpu_sc as plsc

NC = 2
NS = 16
NW = NC * NS
L = 16

V = 1000000
D = 64
B = 16384
TILE_E = 128
NTJ = V // TILE_E
TAIL_BASE = NTJ * TILE_E
TAIL_N = V - TAIL_BASE
TPW = 248
G = 4
RING = 16


def _scalar(x):
  return jnp.max(x)


def _bcast_lane(vec, pos_v):
  dnums = lax.GatherDimensionNumbers(
      offset_dims=(), collapsed_slice_dims=(0,), start_index_map=(0,))
  return lax.gather(
      vec, pos_v.reshape(L, 1), dnums, (1,),
      mode=lax.GatherScatterMode.PROMISE_IN_BOUNDS)


def _emit_extract(tb, is_tail, wlo_s, nj, he, hb, idx_cap, stage, cnt_ref,
                  out_hbm, sem_o, dummy_v):
  wlo_v = jnp.broadcast_to(wlo_s, (L,))
  whi_v = jnp.broadcast_to(wlo_s + nj, (L,))

  def group_body(g, _):
    ev = he[pl.ds(g * L, L)]
    tj = lax.shift_right_logical(ev, 7)
    m = (tj >= wlo_v) & (tj < whi_v)

    @pl.when(jnp.any(m))
    def _():
      bv = hb[pl.ds(g * L, L)]

      def w_cond(carry):
        mm = carry
        return jnp.any(mm)

      def w_body(carry):
        mm = carry
        pos_s = _scalar(plsc.all_reduce_ffs(mm))
        pos_v = jnp.broadcast_to(pos_s, (L,))
        e_all = _bcast_lane(ev, pos_v)
        b_all = _bcast_lane(bv, pos_v)
        if is_tail:
          el = e_all - TAIL_BASE
        else:
          cj = lax.shift_right_logical(e_all, 7) - wlo_v
          el = e_all & (TILE_E - 1)
        c_s = _scalar(cnt_ref[...])
        slot = c_s & (RING - 1)

        @pl.when(c_s >= RING)
        def _():
          pltpu.make_async_copy(
              out_hbm.at[pl.ds(0, D)], dummy_v, sem_o).wait()

        slot_v = jnp.broadcast_to(slot * D, (L,))
        for k in range(D // L):
          f_v = lax.iota(jnp.int32, L) + (k * L)
          if is_tail:
            vals = plsc.load_gather(tb, [f_v * TAIL_N + el])
          else:
            vals = plsc.load_gather(tb, [cj, f_v, el])
          plsc.store_scatter(stage, [slot_v + f_v], vals)
        b_s = _scalar(b_all)
        pltpu.async_copy(
            stage.at[pl.ds(slot * D, D)],
            out_hbm.at[pl.ds(b_s * D, D)],
            sem_o,
        )
        cnt_ref[...] = jnp.broadcast_to(c_s + 1, (L,))
        lane = lax.iota(jnp.int32, L)
        return mm & (lane != jnp.broadcast_to(pos_s, (L,)))

      lax.while_loop(w_cond, w_body, m)

    return 0

  n_groups = lax.div(idx_cap + (L - 1), L)
  lax.fori_loop(0, n_groups, group_body, 0)


def _make_kernel():
  mesh = plsc.VectorSubcoreMesh(core_axis_name="c", subcore_axis_name="s")
  hit_cap = B + L

  @functools.partial(
      pl.kernel,
      mesh=mesh,
      out_type=jax.ShapeDtypeStruct((B * D,), jnp.float32),
      scratch_types=[
          pltpu.VMEM((B,), jnp.int32),
          pltpu.VMEM((hit_cap,), jnp.int32),
          pltpu.VMEM((hit_cap,), jnp.int32),
          pltpu.VMEM((G, D, TILE_E), jnp.float32),
          pltpu.VMEM((G, D, TILE_E), jnp.float32),
          pltpu.VMEM((D * TAIL_N,), jnp.float32),
          pltpu.VMEM((RING * D,), jnp.float32),
          pltpu.VMEM((D,), jnp.float32),
          pltpu.VMEM((L,), jnp.int32),
          pltpu.VMEM((L,), jnp.int32),
          pltpu.SemaphoreType.DMA,
          pltpu.SemaphoreType.DMA,
          pltpu.SemaphoreType.DMA,
      ],
      compiler_params=pltpu.CompilerParams(needs_layout_passes=False),
  )
  def k(tab_hbm, tail_hbm, idx_hbm, out_hbm, idx_v, he, hb, wb0, wb1, tailb,
        stage, dummy_v, pos_ref, cnt_ref, sem_i, sem_t, sem_o):
    wid = lax.axis_index("s") * NC + lax.axis_index("c")
    lo = wid * TPW
    nloc = jnp.minimum(TPW, NTJ - lo)
    is_last = wid == NW - 1
    hi = jnp.where(is_last, NTJ + 1, lo + TPW)
    lo_v = jnp.broadcast_to(lo, (L,))
    hi_v = jnp.broadcast_to(hi, (L,))

    pltpu.sync_copy(idx_hbm, idx_v)
    pos_ref[...] = jnp.zeros((L,), jnp.int32)
    cnt_ref[...] = jnp.zeros((L,), jnp.int32)

    def fill_body(i, _):
      he[pl.ds(i * L, L)] = jnp.broadcast_to(jnp.int32(-1), (L,))
      return 0

    lax.fori_loop(0, hit_cap // L, fill_body, 0)

    def scan_body(i, _):
      v = idx_v[pl.ds(i * L, L)]
      tj = lax.shift_right_logical(v, 7)
      m = (tj >= lo_v) & (tj < hi_v)
      pos_v = pos_ref[...]
      c = plsc.cumsum(m.astype(jnp.int32))
      target = pos_v + c - 1
      plsc.store_scatter(he, [target], v, mask=m)
      bvec = lax.iota(jnp.int32, L) + (i * L)
      plsc.store_scatter(hb, [target], bvec, mask=m)
      pos_ref[...] = pos_v + jnp.broadcast_to(_scalar(c), (L,))
      return 0

    lax.fori_loop(0, B // L, scan_body, 0)
    idx_cap = _scalar(pos_ref[...])

    nwin = lax.div(nloc, G)

    @pl.when(nwin > 0)
    def _():
      for c in range(G):
        pltpu.async_copy(
            tab_hbm.at[:, pl.ds((lo + c) * TILE_E, TILE_E)], wb0.at[c], sem_t)

    def outer_body(o, _):
      for kk in range(2):
        i = o * 2 + kk
        wb_cur = wb0 if kk == 0 else wb1
        wb_nxt = wb1 if kk == 0 else wb0

        @pl.when(i < nwin)
        def _():
          @pl.when(i + 1 < nwin)
          def _():
            for c in range(G):
              pltpu.async_copy(
                  tab_hbm.at[:, pl.ds((lo + (i + 1) * G + c) * TILE_E, TILE_E)],
                  wb_nxt.at[c], sem_t)

          for c in range(G):
            pltpu.make_async_copy(
                tab_hbm.at[:, pl.ds(0, TILE_E)], wb_cur.at[c], sem_t).wait()
          _emit_extract(wb_cur, False, lo + i * G, G, he, hb, idx_cap, stage,
                        cnt_ref, out_hbm, sem_o, dummy_v)

      return 0

    lax.fori_loop(0, (TPW // G + 1) // 2, outer_body, 0)

    @pl.when(is_last)
    def _():
      pltpu.async_copy(tail_hbm, tailb, sem_t)
      pltpu.make_async_copy(tail_hbm, tailb, sem_t).wait()
      _emit_extract(tailb, True, jnp.int32(NTJ), 1, he, hb, idx_cap, stage,
                    cnt_ref, out_hbm, sem_o, dummy_v)

    c_tot = _scalar(cnt_ref[...])

    def drain_body(i, _):
      pltpu.make_async_copy(out_hbm.at[pl.ds(0, D)], dummy_v, sem_o).wait()
      return 0

    lax.fori_loop(0, jnp.minimum(c_tot, RING), drain_body, 0)

  return k


def kernel(index, head_e):
  tab_t = head_e.T
  tail = head_e[TAIL_BASE:].T.reshape(D * TAIL_N)
  flat = _make_kernel()(tab_t, tail, index.astype(jnp.int32))
  return flat.reshape(B, D)


# speedup vs baseline: 3.4401x; 1.0043x over previous
import functools

import jax
import jax.numpy as jnp
from jax import lax
from jax.experimental import pallas as pl
from jax.experimental.pallas import tpu as pltpu
from jax.experimental.pallas import tpu_sc as plsc

NC = 2
NS = 16
NW = NC * NS
L = 16

V = 1000000
D = 64
B = 16384
TILE_E = 128
NTJ = V // TILE_E
TAIL_BASE = NTJ * TILE_E
TAIL_N = V - TAIL_BASE
TPW = 248
G = 4
RING = 16


def _scalar(x):
  return jnp.max(x)


def _bcast_lane(vec, pos_v):
  dnums = lax.GatherDimensionNumbers(
      offset_dims=(), collapsed_slice_dims=(0,), start_index_map=(0,))
  return lax.gather(
      vec, pos_v.reshape(L, 1), dnums, (1,),
      mode=lax.GatherScatterMode.PROMISE_IN_BOUNDS)


def _emit_extract(tb, is_tail, wlo_s, nj, he, hb, idx_cap, stage, cnt_ref,
                  out_hbm, sem_o, dummy_v):
  wlo_v = jnp.broadcast_to(wlo_s, (L,))
  whi_v = jnp.broadcast_to(wlo_s + nj, (L,))

  def group_body(g, _):
    ev = he[pl.ds(g * L, L)]
    tj = lax.shift_right_logical(ev, 7)
    m = (tj >= wlo_v) & (tj < whi_v)

    @pl.when(jnp.any(m))
    def _():
      bv = hb[pl.ds(g * L, L)]

      def w_cond(carry):
        mm = carry
        return jnp.any(mm)

      def w_body(carry):
        mm = carry
        pos_s = _scalar(plsc.all_reduce_ffs(mm))
        pos_v = jnp.broadcast_to(pos_s, (L,))
        e_all = _bcast_lane(ev, pos_v)
        b_all = _bcast_lane(bv, pos_v)
        if is_tail:
          el = e_all - TAIL_BASE
        else:
          cj = lax.shift_right_logical(e_all, 7) - wlo_v
          el = e_all & (TILE_E - 1)
        c_s = _scalar(cnt_ref[...])
        slot = c_s & (RING - 1)

        @pl.when(c_s >= RING)
        def _():
          pltpu.make_async_copy(
              out_hbm.at[pl.ds(0, D)], dummy_v, sem_o).wait()

        slot_v = jnp.broadcast_to(slot * D, (L,))
        for k in range(D // L):
          f_v = lax.iota(jnp.int32, L) + (k * L)
          if is_tail:
            vals = plsc.load_gather(tb, [f_v * TAIL_N + el])
          else:
            vals = plsc.load_gather(tb, [cj, f_v, el])
          plsc.store_scatter(stage, [slot_v + f_v], vals)
        b_s = _scalar(b_all)
        pltpu.async_copy(
            stage.at[pl.ds(slot * D, D)],
            out_hbm.at[pl.ds(b_s * D, D)],
            sem_o,
        )
        cnt_ref[...] = jnp.broadcast_to(c_s + 1, (L,))
        lane = lax.iota(jnp.int32, L)
        return mm & (lane != jnp.broadcast_to(pos_s, (L,)))

      lax.while_loop(w_cond, w_body, m)

    return 0

  n_groups = lax.div(idx_cap + (L - 1), L)
  lax.fori_loop(0, n_groups, group_body, 0)


def _make_kernel():
  mesh = plsc.VectorSubcoreMesh(core_axis_name="c", subcore_axis_name="s")
  hit_cap = B + L

  @functools.partial(
      pl.kernel,
      mesh=mesh,
      out_type=jax.ShapeDtypeStruct((B * D,), jnp.float32),
      scratch_types=[
          pltpu.VMEM((B,), jnp.int32),
          pltpu.VMEM((hit_cap,), jnp.int32),
          pltpu.VMEM((hit_cap,), jnp.int32),
          pltpu.VMEM((G, D, TILE_E), jnp.float32),
          pltpu.VMEM((G, D, TILE_E), jnp.float32),
          pltpu.VMEM((D * TAIL_N,), jnp.float32),
          pltpu.VMEM((RING * D,), jnp.float32),
          pltpu.VMEM((D,), jnp.float32),
          pltpu.VMEM((L,), jnp.int32),
          pltpu.VMEM((L,), jnp.int32),
          pltpu.SemaphoreType.DMA,
          pltpu.SemaphoreType.DMA,
          pltpu.SemaphoreType.DMA,
      ],
      compiler_params=pltpu.CompilerParams(needs_layout_passes=False),
  )
  def k(tab_hbm, tail_hbm, idx_hbm, out_hbm, idx_v, he, hb, wb0, wb1, tailb,
        stage, dummy_v, pos_ref, cnt_ref, sem_i, sem_t, sem_o):
    wid = lax.axis_index("s") * NC + lax.axis_index("c")
    lo = wid * TPW
    nloc = jnp.minimum(TPW, NTJ - lo)
    is_last = wid == NW - 1
    hi = jnp.where(is_last, NTJ + 1, lo + TPW)
    lo_v = jnp.broadcast_to(lo, (L,))
    hi_v = jnp.broadcast_to(hi, (L,))

    nwin_pre = lax.div(nloc, G)

    def fire_win(w, buf):
      for c in range(G):
        pltpu.async_copy(
            tab_hbm.at[:, pl.ds((lo + w * G + c) * TILE_E, TILE_E)],
            buf.at[c], sem_t)

    @pl.when(nwin_pre > 0)
    def _():
      fire_win(0, wb0)

    @pl.when(nwin_pre > 1)
    def _():
      fire_win(1, wb1)

    pltpu.sync_copy(idx_hbm, idx_v)
    pos_ref[...] = jnp.zeros((L,), jnp.int32)
    cnt_ref[...] = jnp.zeros((L,), jnp.int32)

    def fill_body(i, _):
      he[pl.ds(i * L, L)] = jnp.broadcast_to(jnp.int32(-1), (L,))
      return 0

    lax.fori_loop(0, hit_cap // L, fill_body, 0)

    def scan_body(i, _):
      v = idx_v[pl.ds(i * L, L)]
      tj = lax.shift_right_logical(v, 7)
      m = (tj >= lo_v) & (tj < hi_v)
      pos_v = pos_ref[...]
      c = plsc.cumsum(m.astype(jnp.int32))
      target = pos_v + c - 1
      plsc.store_scatter(he, [target], v, mask=m)
      bvec = lax.iota(jnp.int32, L) + (i * L)
      plsc.store_scatter(hb, [target], bvec, mask=m)
      pos_ref[...] = pos_v + jnp.broadcast_to(_scalar(c), (L,))
      return 0

    lax.fori_loop(0, B // L, scan_body, 0)
    idx_cap = _scalar(pos_ref[...])

    nwin = nwin_pre

    def outer_body(o, _):
      for kk in range(2):
        i = o * 2 + kk
        wb_cur = wb0 if kk == 0 else wb1

        @pl.when(i < nwin)
        def _():
          for c in range(G):
            pltpu.make_async_copy(
                tab_hbm.at[:, pl.ds(0, TILE_E)], wb_cur.at[c], sem_t).wait()
          _emit_extract(wb_cur, False, lo + i * G, G, he, hb, idx_cap, stage,
                        cnt_ref, out_hbm, sem_o, dummy_v)

          @pl.when(i + 2 < nwin)
          def _():
            fire_win(i + 2, wb_cur)

      return 0

    lax.fori_loop(0, (TPW // G + 1) // 2, outer_body, 0)

    @pl.when(is_last)
    def _():
      pltpu.async_copy(tail_hbm, tailb, sem_t)
      pltpu.make_async_copy(tail_hbm, tailb, sem_t).wait()
      _emit_extract(tailb, True, jnp.int32(NTJ), 1, he, hb, idx_cap, stage,
                    cnt_ref, out_hbm, sem_o, dummy_v)

    c_tot = _scalar(cnt_ref[...])

    def drain_body(i, _):
      pltpu.make_async_copy(out_hbm.at[pl.ds(0, D)], dummy_v, sem_o).wait()
      return 0

    lax.fori_loop(0, jnp.minimum(c_tot, RING), drain_body, 0)

  return k


def kernel(index, head_e):
  tab_t = head_e.T
  tail = head_e[TAIL_BASE:].T.reshape(D * TAIL_N)
  flat = _make_kernel()(tab_t, tail, index.astype(jnp.int32))
  return flat.reshape(B, D)


# speedup vs baseline: 3.4484x; 1.0024x over previous
import functools

import jax
import jax.numpy as jnp
from jax import lax
from jax.experimental import pallas as pl
from jax.experimental.pallas import tpu as pltpu
from jax.experimental.pallas import tpu_sc as plsc

NC = 2
NS = 16
NW = NC * NS
L = 16

V = 1000000
D = 64
B = 16384
TILE_E = 128
NTJ = V // TILE_E
TAIL_BASE = NTJ * TILE_E
TAIL_N = V - TAIL_BASE
TPW = 248
G = 4
RING = 16


def _scalar(x):
  return jnp.max(x)


def _bcast_lane(vec, pos_v):
  dnums = lax.GatherDimensionNumbers(
      offset_dims=(), collapsed_slice_dims=(0,), start_index_map=(0,))
  return lax.gather(
      vec, pos_v.reshape(L, 1), dnums, (1,),
      mode=lax.GatherScatterMode.PROMISE_IN_BOUNDS)


def _emit_extract(tb, is_tail, wlo_s, nj, he, idx_cap, stage, cnt_ref,
                  out_hbm, sem_o, dummy_v):
  wlo_v = jnp.broadcast_to(wlo_s, (L,))
  whi_v = jnp.broadcast_to(wlo_s + nj, (L,))

  def group_body(g, _):
    ev = he[pl.ds(g * L, L)]
    tj = lax.shift_right_logical(ev, 21)
    m = (tj >= wlo_v) & (tj < whi_v)

    @pl.when(jnp.any(m))
    def _():
      def w_cond(carry):
        mm = carry
        return jnp.any(mm)

      def w_body(carry):
        mm = carry
        pos_s = _scalar(plsc.all_reduce_ffs(mm))
        pos_v = jnp.broadcast_to(pos_s, (L,))
        p_all = _bcast_lane(ev, pos_v)
        b_all = p_all & ((1 << 14) - 1)
        el = lax.shift_right_logical(p_all, 14) & (TILE_E - 1)
        if not is_tail:
          cj = lax.shift_right_logical(p_all, 21) - wlo_v
        c_s = _scalar(cnt_ref[...])
        slot = c_s & (RING - 1)

        @pl.when(c_s >= RING)
        def _():
          pltpu.make_async_copy(
              out_hbm.at[pl.ds(0, D)], dummy_v, sem_o).wait()

        slot_v = jnp.broadcast_to(slot * D, (L,))
        for k in range(D // L):
          f_v = lax.iota(jnp.int32, L) + (k * L)
          if is_tail:
            vals = plsc.load_gather(tb, [f_v * TAIL_N + el])
          else:
            vals = plsc.load_gather(tb, [cj, f_v, el])
          plsc.store_scatter(stage, [slot_v + f_v], vals)
        b_s = _scalar(b_all)
        pltpu.async_copy(
            stage.at[pl.ds(slot * D, D)],
            out_hbm.at[pl.ds(b_s * D, D)],
            sem_o,
        )
        cnt_ref[...] = jnp.broadcast_to(c_s + 1, (L,))
        lane = lax.iota(jnp.int32, L)
        return mm & (lane != jnp.broadcast_to(pos_s, (L,)))

      lax.while_loop(w_cond, w_body, m)

    return 0

  n_groups = lax.div(idx_cap + (L - 1), L)
  lax.fori_loop(0, n_groups, group_body, 0)


def _make_kernel():
  mesh = plsc.VectorSubcoreMesh(core_axis_name="c", subcore_axis_name="s")
  hit_cap = B + L

  @functools.partial(
      pl.kernel,
      mesh=mesh,
      out_type=jax.ShapeDtypeStruct((B * D,), jnp.float32),
      scratch_types=[
          pltpu.VMEM((B,), jnp.int32),
          pltpu.VMEM((hit_cap,), jnp.int32),
          pltpu.VMEM((G, D, TILE_E), jnp.float32),
          pltpu.VMEM((G, D, TILE_E), jnp.float32),
          pltpu.VMEM((D * TAIL_N,), jnp.float32),
          pltpu.VMEM((RING * D,), jnp.float32),
          pltpu.VMEM((D,), jnp.float32),
          pltpu.VMEM((L,), jnp.int32),
          pltpu.VMEM((L,), jnp.int32),
          pltpu.SemaphoreType.DMA,
          pltpu.SemaphoreType.DMA,
          pltpu.SemaphoreType.DMA,
      ],
      compiler_params=pltpu.CompilerParams(needs_layout_passes=False),
  )
  def k(tab_hbm, tail_hbm, idx_hbm, out_hbm, idx_v, he, wb0, wb1, tailb,
        stage, dummy_v, pos_ref, cnt_ref, sem_i, sem_t, sem_o):
    wid = lax.axis_index("s") * NC + lax.axis_index("c")
    lo = wid * TPW
    nloc = jnp.minimum(TPW, NTJ - lo)
    is_last = wid == NW - 1
    hi = jnp.where(is_last, NTJ + 1, lo + TPW)
    lo_v = jnp.broadcast_to(lo, (L,))
    hi_v = jnp.broadcast_to(hi, (L,))

    nwin_pre = lax.div(nloc, G)

    def fire_win(w, buf):
      for c in range(G):
        pltpu.async_copy(
            tab_hbm.at[:, pl.ds((lo + w * G + c) * TILE_E, TILE_E)],
            buf.at[c], sem_t)

    @pl.when(nwin_pre > 0)
    def _():
      fire_win(0, wb0)

    @pl.when(nwin_pre > 1)
    def _():
      fire_win(1, wb1)

    pltpu.sync_copy(idx_hbm, idx_v)
    pos_ref[...] = jnp.zeros((L,), jnp.int32)
    cnt_ref[...] = jnp.zeros((L,), jnp.int32)

    def fill_body(i, _):
      he[pl.ds(i * L, L)] = jnp.broadcast_to(jnp.int32(-1), (L,))
      return 0

    lax.fori_loop(0, hit_cap // L, fill_body, 0)

    def scan_body(i, _):
      v = idx_v[pl.ds(i * L, L)]
      tj = lax.shift_right_logical(v, 7)
      m = (tj >= lo_v) & (tj < hi_v)
      pos_v = pos_ref[...]
      c = plsc.cumsum(m.astype(jnp.int32))
      target = pos_v + c - 1
      bvec = lax.iota(jnp.int32, L) + (i * L)
      packed = (lax.shift_left(tj - lo_v, 21)
                | lax.shift_left(v & (TILE_E - 1), 14) | bvec)
      plsc.store_scatter(he, [target], packed, mask=m)
      pos_ref[...] = pos_v + jnp.broadcast_to(_scalar(c), (L,))
      return 0

    lax.fori_loop(0, B // L, scan_body, 0)
    idx_cap = _scalar(pos_ref[...])

    nwin = nwin_pre

    def outer_body(o, _):
      for kk in range(2):
        i = o * 2 + kk
        wb_cur = wb0 if kk == 0 else wb1

        @pl.when(i < nwin)
        def _():
          for c in range(G):
            pltpu.make_async_copy(
                tab_hbm.at[:, pl.ds(0, TILE_E)], wb_cur.at[c], sem_t).wait()
          _emit_extract(wb_cur, False, i * G, G, he, idx_cap, stage,
                        cnt_ref, out_hbm, sem_o, dummy_v)

          @pl.when(i + 2 < nwin)
          def _():
            fire_win(i + 2, wb_cur)

      return 0

    lax.fori_loop(0, (TPW // G + 1) // 2, outer_body, 0)

    @pl.when(is_last)
    def _():
      pltpu.async_copy(tail_hbm, tailb, sem_t)
      pltpu.make_async_copy(tail_hbm, tailb, sem_t).wait()
      _emit_extract(tailb, True, NTJ - lo, 1, he, idx_cap, stage,
                    cnt_ref, out_hbm, sem_o, dummy_v)

    c_tot = _scalar(cnt_ref[...])

    def drain_body(i, _):
      pltpu.make_async_copy(out_hbm.at[pl.ds(0, D)], dummy_v, sem_o).wait()
      return 0

    lax.fori_loop(0, jnp.minimum(c_tot, RING), drain_body, 0)

  return k


def kernel(index, head_e):
  tab_t = head_e.T
  tail = head_e[TAIL_BASE:].T.reshape(D * TAIL_N)
  flat = _make_kernel()(tab_t, tail, index.astype(jnp.int32))
  return flat.reshape(B, D)


# speedup vs baseline: 3.7805x; 1.0963x over previous
import functools

import jax
import jax.numpy as jnp
from jax import lax
from jax.experimental import pallas as pl
from jax.experimental.pallas import tpu as pltpu
from jax.experimental.pallas import tpu_sc as plsc

NC = 2
NS = 16
NW = NC * NS
L = 16

V = 1000000
D = 64
B = 16384
TILE_E = 128
NTJ = V // TILE_E
TAIL_BASE = NTJ * TILE_E
TAIL_N = V - TAIL_BASE
TPW = 248
G = 4
RING = 16


def _scalar(x):
  return jnp.max(x)


def _bcast_lane(vec, pos_v):
  dnums = lax.GatherDimensionNumbers(
      offset_dims=(), collapsed_slice_dims=(0,), start_index_map=(0,))
  return lax.gather(
      vec, pos_v.reshape(L, 1), dnums, (1,),
      mode=lax.GatherScatterMode.PROMISE_IN_BOUNDS)


def _emit_extract(tb, is_tail, wlo_s, nj, he, idx_cap, stage, cnt_ref,
                  out_hbm, sem_o, dummy_v):
  wlo_v = jnp.broadcast_to(wlo_s, (L,))
  whi_v = jnp.broadcast_to(wlo_s + nj, (L,))

  def group_body(g4, _):
    evs, ms = [], []
    for k in range(4):
      ev_k = he[pl.ds(g4 * (4 * L) + k * L, L)]
      tj_k = lax.shift_right_logical(ev_k, 21)
      evs.append(ev_k)
      ms.append((tj_k >= wlo_v) & (tj_k < whi_v))
    any4 = (ms[0] | ms[1]) | (ms[2] | ms[3])

    @pl.when(jnp.any(any4))
    def _():
      for k in range(4):
        _drill(evs[k], ms[k], is_tail, wlo_v, stage, cnt_ref, out_hbm, sem_o,
               dummy_v, tb)
    return 0

  n_groups = lax.div(idx_cap + (4 * L - 1), 4 * L)
  lax.fori_loop(0, n_groups, group_body, 0)


def _drill(ev, m, is_tail, wlo_v, stage, cnt_ref, out_hbm, sem_o, dummy_v, tb):
  @pl.when(jnp.any(m))
  def _():
      def w_cond(carry):
        mm = carry
        return jnp.any(mm)

      def w_body(carry):
        mm = carry
        pos_s = _scalar(plsc.all_reduce_ffs(mm))
        pos_v = jnp.broadcast_to(pos_s, (L,))
        p_all = _bcast_lane(ev, pos_v)
        b_all = p_all & ((1 << 14) - 1)
        el = lax.shift_right_logical(p_all, 14) & (TILE_E - 1)
        if not is_tail:
          cj = lax.shift_right_logical(p_all, 21) - wlo_v
        c_s = _scalar(cnt_ref[...])
        slot = c_s & (RING - 1)

        @pl.when(c_s >= RING)
        def _():
          pltpu.make_async_copy(
              out_hbm.at[pl.ds(0, D)], dummy_v, sem_o).wait()

        slot_v = jnp.broadcast_to(slot * D, (L,))
        for k in range(D // L):
          f_v = lax.iota(jnp.int32, L) + (k * L)
          if is_tail:
            vals = plsc.load_gather(tb, [f_v * TAIL_N + el])
          else:
            vals = plsc.load_gather(tb, [cj, f_v, el])
          plsc.store_scatter(stage, [slot_v + f_v], vals)
        b_s = _scalar(b_all)
        pltpu.async_copy(
            stage.at[pl.ds(slot * D, D)],
            out_hbm.at[pl.ds(b_s * D, D)],
            sem_o,
        )
        cnt_ref[...] = jnp.broadcast_to(c_s + 1, (L,))
        lane = lax.iota(jnp.int32, L)
        return mm & (lane != jnp.broadcast_to(pos_s, (L,)))

      lax.while_loop(w_cond, w_body, m)


def _make_kernel():
  mesh = plsc.VectorSubcoreMesh(core_axis_name="c", subcore_axis_name="s")
  hit_cap = B + 6 * L

  @functools.partial(
      pl.kernel,
      mesh=mesh,
      out_type=jax.ShapeDtypeStruct((B * D,), jnp.float32),
      scratch_types=[
          pltpu.VMEM((B,), jnp.int32),
          pltpu.VMEM((hit_cap,), jnp.int32),
          pltpu.VMEM((G, D, TILE_E), jnp.float32),
          pltpu.VMEM((G, D, TILE_E), jnp.float32),
          pltpu.VMEM((D * TAIL_N,), jnp.float32),
          pltpu.VMEM((RING * D,), jnp.float32),
          pltpu.VMEM((D,), jnp.float32),
          pltpu.VMEM((L,), jnp.int32),
          pltpu.VMEM((L,), jnp.int32),
          pltpu.SemaphoreType.DMA,
          pltpu.SemaphoreType.DMA,
          pltpu.SemaphoreType.DMA,
      ],
      compiler_params=pltpu.CompilerParams(needs_layout_passes=False),
  )
  def k(tab_hbm, tail_hbm, idx_hbm, out_hbm, idx_v, he, wb0, wb1, tailb,
        stage, dummy_v, pos_ref, cnt_ref, sem_i, sem_t, sem_o):
    wid = lax.axis_index("s") * NC + lax.axis_index("c")
    lo = wid * TPW
    nloc = jnp.minimum(TPW, NTJ - lo)
    is_last = wid == NW - 1
    hi = jnp.where(is_last, NTJ + 1, lo + TPW)
    lo_v = jnp.broadcast_to(lo, (L,))
    hi_v = jnp.broadcast_to(hi, (L,))

    nwin_pre = lax.div(nloc, G)

    def fire_win(w, buf):
      for c in range(G):
        pltpu.async_copy(
            tab_hbm.at[:, pl.ds((lo + w * G + c) * TILE_E, TILE_E)],
            buf.at[c], sem_t)

    @pl.when(nwin_pre > 0)
    def _():
      fire_win(0, wb0)

    @pl.when(nwin_pre > 1)
    def _():
      fire_win(1, wb1)

    pltpu.sync_copy(idx_hbm, idx_v)
    pos_ref[...] = jnp.zeros((L,), jnp.int32)
    cnt_ref[...] = jnp.zeros((L,), jnp.int32)

    def scan_body(i, _):
      pos_v = pos_ref[...]
      iot = lax.iota(jnp.int32, L)
      for k in range(2):
        v = idx_v[pl.ds(i * (2 * L) + k * L, L)]
        tj = lax.shift_right_logical(v, 7)
        m = (tj >= lo_v) & (tj < hi_v)
        c = plsc.cumsum(m.astype(jnp.int32))
        target = pos_v + c - 1
        bvec = iot + (i * (2 * L) + k * L)
        packed = (lax.shift_left(tj - lo_v, 21)
                  | lax.shift_left(v & (TILE_E - 1), 14) | bvec)
        plsc.store_scatter(he, [target], packed, mask=m)
        pos_v = pos_v + jnp.broadcast_to(_scalar(c), (L,))
      pos_ref[...] = pos_v
      return 0

    lax.fori_loop(0, B // (2 * L), scan_body, 0)
    idx_cap = _scalar(pos_ref[...])
    cap_v = jnp.broadcast_to(idx_cap, (L,))
    neg1 = jnp.broadcast_to(jnp.int32(-1), (L,))
    iot_p = lax.iota(jnp.int32, L)
    for k in range(5):
      plsc.store_scatter(he, [cap_v + iot_p + k * L], neg1)

    nwin = nwin_pre

    def outer_body(o, _):
      for kk in range(2):
        i = o * 2 + kk
        wb_cur = wb0 if kk == 0 else wb1

        @pl.when(i < nwin)
        def _():
          for c in range(G):
            pltpu.make_async_copy(
                tab_hbm.at[:, pl.ds(0, TILE_E)], wb_cur.at[c], sem_t).wait()
          _emit_extract(wb_cur, False, i * G, G, he, idx_cap, stage,
                        cnt_ref, out_hbm, sem_o, dummy_v)

          @pl.when(i + 2 < nwin)
          def _():
            fire_win(i + 2, wb_cur)

      return 0

    lax.fori_loop(0, (TPW // G + 1) // 2, outer_body, 0)

    @pl.when(is_last)
    def _():
      pltpu.async_copy(tail_hbm, tailb, sem_t)
      pltpu.make_async_copy(tail_hbm, tailb, sem_t).wait()
      _emit_extract(tailb, True, NTJ - lo, 1, he, idx_cap, stage,
                    cnt_ref, out_hbm, sem_o, dummy_v)

    c_tot = _scalar(cnt_ref[...])

    def drain_body(i, _):
      pltpu.make_async_copy(out_hbm.at[pl.ds(0, D)], dummy_v, sem_o).wait()
      return 0

    lax.fori_loop(0, jnp.minimum(c_tot, RING), drain_body, 0)

  return k


def kernel(index, head_e):
  tab_t = head_e.T
  tail = head_e[TAIL_BASE:].T.reshape(D * TAIL_N)
  flat = _make_kernel()(tab_t, tail, index.astype(jnp.int32))
  return flat.reshape(B, D)


# speedup vs baseline: 4.1513x; 1.0981x over previous
import functools

import jax
import jax.numpy as jnp
from jax import lax
from jax.experimental import pallas as pl
from jax.experimental.pallas import tpu as pltpu
from jax.experimental.pallas import tpu_sc as plsc

NC = 2
NS = 16
NW = NC * NS
L = 16

V = 1000000
D = 64
B = 16384
TILE_E = 128
NTJ = V // TILE_E
TAIL_BASE = NTJ * TILE_E
TAIL_N = V - TAIL_BASE
TPW = 246
G = 6
RING = 16


def _scalar(x):
  return jnp.max(x)


def _bcast_lane(vec, pos_v):
  dnums = lax.GatherDimensionNumbers(
      offset_dims=(), collapsed_slice_dims=(0,), start_index_map=(0,))
  return lax.gather(
      vec, pos_v.reshape(L, 1), dnums, (1,),
      mode=lax.GatherScatterMode.PROMISE_IN_BOUNDS)


def _emit_extract(tb, is_tail, wlo_s, nj, he, idx_cap, stage, cnt_ref,
                  out_hbm, sem_o, dummy_v):
  wlo_v = jnp.broadcast_to(wlo_s, (L,))
  whi_v = jnp.broadcast_to(wlo_s + nj, (L,))

  def group_body(g4, _):
    evs, ms = [], []
    for k in range(4):
      ev_k = he[pl.ds(g4 * (4 * L) + k * L, L)]
      tj_k = lax.shift_right_logical(ev_k, 21)
      evs.append(ev_k)
      ms.append((tj_k >= wlo_v) & (tj_k < whi_v))
    any4 = (ms[0] | ms[1]) | (ms[2] | ms[3])

    @pl.when(jnp.any(any4))
    def _():
      for k in range(4):
        _drill(evs[k], ms[k], is_tail, wlo_v, stage, cnt_ref, out_hbm, sem_o,
               dummy_v, tb)
    return 0

  n_groups = lax.div(idx_cap + (4 * L - 1), 4 * L)
  lax.fori_loop(0, n_groups, group_body, 0)


def _drill(ev, m, is_tail, wlo_v, stage, cnt_ref, out_hbm, sem_o, dummy_v, tb):
  @pl.when(jnp.any(m))
  def _():
      def w_cond(carry):
        mm = carry
        return jnp.any(mm)

      def w_body(carry):
        mm = carry
        pos_s = _scalar(plsc.all_reduce_ffs(mm))
        pos_v = jnp.broadcast_to(pos_s, (L,))
        p_all = _bcast_lane(ev, pos_v)
        b_all = p_all & ((1 << 14) - 1)
        el = lax.shift_right_logical(p_all, 14) & (TILE_E - 1)
        if not is_tail:
          cj = lax.shift_right_logical(p_all, 21) - wlo_v
        c_s = _scalar(cnt_ref[...])
        slot = c_s & (RING - 1)

        @pl.when(c_s >= RING)
        def _():
          pltpu.make_async_copy(
              out_hbm.at[pl.ds(0, D)], dummy_v, sem_o).wait()

        slot_v = jnp.broadcast_to(slot * D, (L,))
        for k in range(D // L):
          f_v = lax.iota(jnp.int32, L) + (k * L)
          if is_tail:
            vals = plsc.load_gather(tb, [f_v * TAIL_N + el])
          else:
            vals = plsc.load_gather(tb, [cj, f_v, el])
          plsc.store_scatter(stage, [slot_v + f_v], vals)
        b_s = _scalar(b_all)
        pltpu.async_copy(
            stage.at[pl.ds(slot * D, D)],
            out_hbm.at[pl.ds(b_s * D, D)],
            sem_o,
        )
        cnt_ref[...] = jnp.broadcast_to(c_s + 1, (L,))
        lane = lax.iota(jnp.int32, L)
        return mm & (lane != jnp.broadcast_to(pos_s, (L,)))

      lax.while_loop(w_cond, w_body, m)


def _make_kernel():
  mesh = plsc.VectorSubcoreMesh(core_axis_name="c", subcore_axis_name="s")
  hit_cap = B + 6 * L

  @functools.partial(
      pl.kernel,
      mesh=mesh,
      out_type=jax.ShapeDtypeStruct((B * D,), jnp.float32),
      scratch_types=[
          pltpu.VMEM((B // 4,), jnp.int32),
          pltpu.VMEM((hit_cap,), jnp.int32),
          pltpu.VMEM((G, D, TILE_E), jnp.float32),
          pltpu.VMEM((G, D, TILE_E), jnp.float32),
          pltpu.VMEM((D * TAIL_N,), jnp.float32),
          pltpu.VMEM((RING * D,), jnp.float32),
          pltpu.VMEM((D,), jnp.float32),
          pltpu.VMEM((L,), jnp.int32),
          pltpu.VMEM((L,), jnp.int32),
          pltpu.SemaphoreType.DMA,
          pltpu.SemaphoreType.DMA,
          pltpu.SemaphoreType.DMA,
      ],
      compiler_params=pltpu.CompilerParams(needs_layout_passes=False),
  )
  def k(tab_hbm, tail_hbm, idx_hbm, out_hbm, idx_v, he, wb0, wb1, tailb,
        stage, dummy_v, pos_ref, cnt_ref, sem_i, sem_t, sem_o):
    wid = lax.axis_index("s") * NC + lax.axis_index("c")
    lo = wid * TPW
    nloc = jnp.minimum(TPW, NTJ - lo)
    is_last = wid == NW - 1
    hi = jnp.where(is_last, NTJ + 1, lo + TPW)
    lo_v = jnp.broadcast_to(lo, (L,))
    hi_v = jnp.broadcast_to(hi, (L,))

    nwin_pre = lax.div(nloc, G)

    def fire_win(w, buf):
      for c in range(G):
        pltpu.async_copy(
            tab_hbm.at[:, pl.ds((lo + w * G + c) * TILE_E, TILE_E)],
            buf.at[c], sem_t)

    @pl.when(nwin_pre > 0)
    def _():
      fire_win(0, wb0)

    @pl.when(nwin_pre > 1)
    def _():
      fire_win(1, wb1)

    pos_ref[...] = jnp.zeros((L,), jnp.int32)
    cnt_ref[...] = jnp.zeros((L,), jnp.int32)

    for ci in range(4):
      pltpu.sync_copy(idx_hbm.at[pl.ds(ci * (B // 4), B // 4)], idx_v)

      def scan_body(i, _, _ci=ci):
        pos_v = pos_ref[...]
        iot = lax.iota(jnp.int32, L)
        for k in range(2):
          v = idx_v[pl.ds(i * (2 * L) + k * L, L)]
          tj = lax.shift_right_logical(v, 7)
          m = (tj >= lo_v) & (tj < hi_v)
          c = plsc.cumsum(m.astype(jnp.int32))
          target = pos_v + c - 1
          bvec = iot + (_ci * (B // 4) + i * (2 * L) + k * L)
          packed = (lax.shift_left(tj - lo_v, 21)
                    | lax.shift_left(v & (TILE_E - 1), 14) | bvec)
          plsc.store_scatter(he, [target], packed, mask=m)
          pos_v = pos_v + jnp.broadcast_to(_scalar(c), (L,))
        pos_ref[...] = pos_v
        return 0

      lax.fori_loop(0, (B // 4) // (2 * L), scan_body, 0)
    idx_cap = _scalar(pos_ref[...])
    cap_v = jnp.broadcast_to(idx_cap, (L,))
    neg1 = jnp.broadcast_to(jnp.int32(-1), (L,))
    iot_p = lax.iota(jnp.int32, L)
    for k in range(5):
      plsc.store_scatter(he, [cap_v + iot_p + k * L], neg1)

    nwin = nwin_pre

    def outer_body(o, _):
      for kk in range(2):
        i = o * 2 + kk
        wb_cur = wb0 if kk == 0 else wb1

        @pl.when(i < nwin)
        def _():
          for c in range(G):
            pltpu.make_async_copy(
                tab_hbm.at[:, pl.ds(0, TILE_E)], wb_cur.at[c], sem_t).wait()
          _emit_extract(wb_cur, False, i * G, G, he, idx_cap, stage,
                        cnt_ref, out_hbm, sem_o, dummy_v)

          @pl.when(i + 2 < nwin)
          def _():
            fire_win(i + 2, wb_cur)

      return 0

    lax.fori_loop(0, (TPW // G + 1) // 2, outer_body, 0)

    @pl.when(is_last)
    def _():
      pltpu.async_copy(tail_hbm, tailb, sem_t)
      pltpu.make_async_copy(tail_hbm, tailb, sem_t).wait()
      _emit_extract(tailb, True, NTJ - lo, 1, he, idx_cap, stage,
                    cnt_ref, out_hbm, sem_o, dummy_v)

    c_tot = _scalar(cnt_ref[...])

    def drain_body(i, _):
      pltpu.make_async_copy(out_hbm.at[pl.ds(0, D)], dummy_v, sem_o).wait()
      return 0

    lax.fori_loop(0, jnp.minimum(c_tot, RING), drain_body, 0)

  return k


def kernel(index, head_e):
  tab_t = head_e.T
  tail = head_e[TAIL_BASE:].T.reshape(D * TAIL_N)
  flat = _make_kernel()(tab_t, tail, index.astype(jnp.int32))
  return flat.reshape(B, D)
